# baseline (speedup 1.0000x reference)
import functools

import jax
import jax.numpy as jnp
from jax.experimental import pallas as pl
from jax.experimental.pallas import tpu as pltpu

_D = 256
_PC_RANGE = (-51.2, -51.2, -5.0, 51.2, 51.2, 3.0)
_INF_PC_RANGE = (-76.8, -76.8, -5.0, 76.8, 76.8, 3.0)

_BIG = 1e12
_ACC_THRESH = 1e10


def _denorm(pts, pr):
    x = pts[:, 0:1] * (pr[3] - pr[0]) + pr[0]
    y = pts[:, 1:2] * (pr[4] - pr[1]) + pr[1]
    z = pts[:, 2:3] * (pr[5] - pr[2]) + pr[2]
    return jnp.concatenate([x, y, z], axis=-1)


def _inv4(m):
    rows = []
    idx = (0, 1, 2, 3)
    for r in idx:
        row = []
        for c in idx:
            rs = [i for i in idx if i != r]
            cs = [j for j in idx if j != c]
            det3 = (
                m[rs[0], cs[0]] * (m[rs[1], cs[1]] * m[rs[2], cs[2]]
                                   - m[rs[1], cs[2]] * m[rs[2], cs[1]])
                - m[rs[0], cs[1]] * (m[rs[1], cs[0]] * m[rs[2], cs[2]]
                                     - m[rs[1], cs[2]] * m[rs[2], cs[0]])
                + m[rs[0], cs[2]] * (m[rs[1], cs[0]] * m[rs[2], cs[1]]
                                     - m[rs[1], cs[1]] * m[rs[2], cs[0]])
            )
            row.append(((-1.0) ** (r + c)) * det3)
        rows.append(jnp.stack(row))
    cof = jnp.stack(rows)
    det = jnp.sum(m[0] * cof[0])
    return cof.T / det


def _align_body(q_ref, wp_ref, wf_ref, wfus_ref, bp_ref, bf_ref, bfus_ref,
                aligned_ref, fused_ref):
    q = q_ref[...]
    pos = jnp.dot(q[:, :_D], wp_ref[...],
                  preferred_element_type=jnp.float32) + bp_ref[...]
    feat = jnp.dot(q[:, _D:], wf_ref[...],
                   preferred_element_type=jnp.float32) + bf_ref[...]
    aligned_ref[:, :_D] = pos
    aligned_ref[:, _D:] = feat
    fused_ref[...] = jnp.dot(feat, wfus_ref[...],
                             preferred_element_type=jnp.float32) + bfus_ref[...]


def _match_body(vr_ref, vd_ref, vs_ref, infT_ref, idx_ref, *, bv, n_inf, n_veh):
    infx = infT_ref[0:1, :]
    infy = infT_ref[1:2, :]
    infz = infT_ref[2:3, :]
    sx = _PC_RANGE[3] - _PC_RANGE[0]
    sy = _PC_RANGE[4] - _PC_RANGE[1]
    sz = _PC_RANGE[5] - _PC_RANGE[2]

    def step(i, carry):
        val, idx = carry
        r0 = i * bv
        vr = vr_ref[pl.ds(r0, bv), :]
        vd = vd_ref[pl.ds(r0, bv), :]
        vs = vs_ref[pl.ds(r0, bv), :]
        ok = jnp.maximum(jnp.maximum(vs[:, 0:1], vs[:, 1:2]), vs[:, 2:3]) >= 0.05
        dx = (vr[:, 0:1] * sx + _PC_RANGE[0]) - infx
        dy = (vr[:, 1:2] * sy + _PC_RANGE[1]) - infy
        dz = (vr[:, 2:3] * sz + _PC_RANGE[2]) - infz
        within = ((jnp.abs(dx) <= jnp.where(ok, vd[:, 0:1], -1.0))
                  & (jnp.abs(dy) <= vd[:, 1:2])
                  & (jnp.abs(dz) <= vd[:, 2:3]))
        dist2 = dx * dx + dy * dy + dz * dz
        cost = jnp.where(within, dist2, _BIG)
        bmin = jnp.min(cost, axis=0, keepdims=True)
        rows = jax.lax.broadcasted_iota(jnp.int32, (bv, n_inf), 0) + r0
        barg = jnp.min(jnp.where(cost == bmin, rows, jnp.int32(2 ** 30)),
                       axis=0, keepdims=True)
        upd = bmin < val
        return (jnp.where(upd, bmin, val), jnp.where(upd, barg, idx))

    val0 = jnp.full((1, n_inf), _BIG, jnp.float32)
    idx0 = jnp.full((1, n_inf), -1, jnp.int32)
    val, idx = jax.lax.fori_loop(0, n_veh // bv, step, (val0, idx0))
    idx_ref[...] = jnp.where(val < _ACC_THRESH, idx, -1)


def _scatter_body(vq_ref, idx_ref, fused_ref, out_ref, *, bv, n_inf):
    i = pl.program_id(0)
    vq = vq_ref[...]
    rows = jax.lax.broadcasted_iota(jnp.int32, (bv, n_inf), 0) + i * bv
    onehot = (rows == idx_ref[...]).astype(jnp.float32)
    contrib = jnp.dot(onehot, fused_ref[...],
                      preferred_element_type=jnp.float32)
    out_ref[:, :_D] = vq[:, :_D]
    out_ref[:, _D:] = vq[:, _D:] + contrib


def kernel(inf_query, inf_reference, veh_query, veh_reference, veh_pred_dims,
           veh_scores, veh2inf_rt, W_align, b_align, W_align_pos, b_align_pos,
           W_fusion, b_fusion):
    n_inf = inf_query.shape[0]
    n_veh = veh_query.shape[0]

    inf_pts = _denorm(inf_reference, _INF_PC_RANGE)
    calib = _inv4(veh2inf_rt[0].T)
    homog = jnp.concatenate([inf_pts, jnp.ones_like(inf_pts[:, :1])], axis=-1)
    inf_ptsT = (homog @ calib.T)[:, :3].T
    r9 = calib[:3, :3].reshape(1, 9)
    bp_eff = r9 @ W_align_pos[_D:] + b_align_pos[None]
    bf_eff = r9 @ W_align[_D:] + b_align[None]

    bq = 512
    aligned, fused = pl.pallas_call(
        _align_body,
        grid=(n_inf // bq,),
        in_specs=[
            pl.BlockSpec((bq, 2 * _D), lambda i: (i, 0)),
            pl.BlockSpec((_D, _D), lambda i: (0, 0)),
            pl.BlockSpec((_D, _D), lambda i: (0, 0)),
            pl.BlockSpec((_D, _D), lambda i: (0, 0)),
            pl.BlockSpec((1, _D), lambda i: (0, 0)),
            pl.BlockSpec((1, _D), lambda i: (0, 0)),
            pl.BlockSpec((1, _D), lambda i: (0, 0)),
        ],
        out_specs=[
            pl.BlockSpec((bq, 2 * _D), lambda i: (i, 0)),
            pl.BlockSpec((bq, _D), lambda i: (i, 0)),
        ],
        out_shape=[
            jax.ShapeDtypeStruct((n_inf, 2 * _D), jnp.float32),
            jax.ShapeDtypeStruct((n_inf, _D), jnp.float32),
        ],
    )(inf_query, W_align_pos[:_D], W_align[:_D], W_fusion,
      bp_eff, bf_eff, b_fusion[None])

    bv = 256
    best_idx = pl.pallas_call(
        functools.partial(_match_body, bv=bv, n_inf=n_inf, n_veh=n_veh),
        grid=(1,),
        in_specs=[
            pl.BlockSpec((n_veh, 3), lambda i: (0, 0)),
            pl.BlockSpec((n_veh, 3), lambda i: (0, 0)),
            pl.BlockSpec((n_veh, 3), lambda i: (0, 0)),
            pl.BlockSpec((3, n_inf), lambda i: (0, 0)),
        ],
        out_specs=pl.BlockSpec((1, n_inf), lambda i: (0, 0)),
        out_shape=jax.ShapeDtypeStruct((1, n_inf), jnp.int32),
    )(veh_reference, veh_pred_dims, veh_scores, inf_ptsT)

    bs = 512
    veh_out = pl.pallas_call(
        functools.partial(_scatter_body, bv=bs, n_inf=n_inf),
        grid=(n_veh // bs,),
        in_specs=[
            pl.BlockSpec((bs, 2 * _D), lambda i: (i, 0)),
            pl.BlockSpec((1, n_inf), lambda i: (0, 0)),
            pl.BlockSpec((n_inf, _D), lambda i: (0, 0)),
        ],
        out_specs=pl.BlockSpec((bs, 2 * _D), lambda i: (i, 0)),
        out_shape=jax.ShapeDtypeStruct((n_veh, 2 * _D), jnp.float32),
    )(veh_query, best_idx, fused)

    return veh_out, aligned


# speedup vs baseline: 1.0016x; 1.0016x over previous
import functools

import jax
import jax.numpy as jnp
from jax.experimental import pallas as pl
from jax.experimental.pallas import tpu as pltpu

_D = 256
_PC_RANGE = (-51.2, -51.2, -5.0, 51.2, 51.2, 3.0)
_INF_PC_RANGE = (-76.8, -76.8, -5.0, 76.8, 76.8, 3.0)

_BIG = 1e12
_ACC_THRESH = 1e10


def _denorm(pts, pr):
    x = pts[:, 0:1] * (pr[3] - pr[0]) + pr[0]
    y = pts[:, 1:2] * (pr[4] - pr[1]) + pr[1]
    z = pts[:, 2:3] * (pr[5] - pr[2]) + pr[2]
    return jnp.concatenate([x, y, z], axis=-1)


def _inv4(m):
    rows = []
    idx = (0, 1, 2, 3)
    for r in idx:
        row = []
        for c in idx:
            rs = [i for i in idx if i != r]
            cs = [j for j in idx if j != c]
            det3 = (
                m[rs[0], cs[0]] * (m[rs[1], cs[1]] * m[rs[2], cs[2]]
                                   - m[rs[1], cs[2]] * m[rs[2], cs[1]])
                - m[rs[0], cs[1]] * (m[rs[1], cs[0]] * m[rs[2], cs[2]]
                                     - m[rs[1], cs[2]] * m[rs[2], cs[0]])
                + m[rs[0], cs[2]] * (m[rs[1], cs[0]] * m[rs[2], cs[1]]
                                     - m[rs[1], cs[1]] * m[rs[2], cs[0]])
            )
            row.append(((-1.0) ** (r + c)) * det3)
        rows.append(jnp.stack(row))
    cof = jnp.stack(rows)
    det = jnp.sum(m[0] * cof[0])
    return cof.T / det


def _align_body(q_ref, wp_ref, wf_ref, wfus_ref, bp_ref, bf_ref, bfus_ref,
                aligned_ref, fused_ref):
    q = q_ref[...]
    pos = jnp.dot(q[:, :_D], wp_ref[...],
                  preferred_element_type=jnp.float32) + bp_ref[...]
    feat = jnp.dot(q[:, _D:], wf_ref[...],
                   preferred_element_type=jnp.float32) + bf_ref[...]
    aligned_ref[:, :_D] = pos
    aligned_ref[:, _D:] = feat
    fused_ref[...] = jnp.dot(feat, wfus_ref[...],
                             preferred_element_type=jnp.float32) + bfus_ref[...]


def _match_body(vr_ref, vd_ref, vs_ref, infT_ref, idx_ref, val_ref,
                *, bv, n_inf):
    i = pl.program_id(0)

    @pl.when(i == 0)
    def _():
        val_ref[...] = jnp.full((1, n_inf), _BIG, jnp.float32)
        idx_ref[...] = jnp.full((1, n_inf), -1, jnp.int32)

    infx = infT_ref[0:1, :]
    infy = infT_ref[1:2, :]
    infz = infT_ref[2:3, :]
    sx = _PC_RANGE[3] - _PC_RANGE[0]
    sy = _PC_RANGE[4] - _PC_RANGE[1]
    sz = _PC_RANGE[5] - _PC_RANGE[2]
    vr = vr_ref[...]
    vd = vd_ref[...]
    vs = vs_ref[...]
    ok = jnp.maximum(jnp.maximum(vs[:, 0:1], vs[:, 1:2]), vs[:, 2:3]) >= 0.05
    dx = (vr[:, 0:1] * sx + _PC_RANGE[0]) - infx
    dy = (vr[:, 1:2] * sy + _PC_RANGE[1]) - infy
    dz = (vr[:, 2:3] * sz + _PC_RANGE[2]) - infz
    within = ((jnp.abs(dx) <= jnp.where(ok, vd[:, 0:1], -1.0))
              & (jnp.abs(dy) <= vd[:, 1:2])
              & (jnp.abs(dz) <= vd[:, 2:3]))
    dist2 = dx * dx + dy * dy + dz * dz
    cost = jnp.where(within, dist2, _BIG)
    bmin = jnp.min(cost, axis=0, keepdims=True)
    rows = jax.lax.broadcasted_iota(jnp.int32, (bv, n_inf), 0) + i * bv
    barg = jnp.min(jnp.where(cost == bmin, rows, jnp.int32(2 ** 30)),
                   axis=0, keepdims=True)
    cur = val_ref[...]
    upd = bmin < cur
    val_ref[...] = jnp.where(upd, bmin, cur)
    idx_ref[...] = jnp.where(upd, barg, idx_ref[...])

    @pl.when(i == pl.num_programs(0) - 1)
    def _():
        idx_ref[...] = jnp.where(val_ref[...] < _ACC_THRESH, idx_ref[...], -1)


def _scatter_body(vq_ref, idx_ref, fused_ref, out_ref, *, bv, n_inf):
    i = pl.program_id(0)
    vq = vq_ref[...]
    rows = jax.lax.broadcasted_iota(jnp.int32, (bv, n_inf), 0) + i * bv
    onehot = (rows == idx_ref[...]).astype(jnp.float32)
    contrib = jnp.dot(onehot, fused_ref[...],
                      preferred_element_type=jnp.float32)
    out_ref[:, :_D] = vq[:, :_D]
    out_ref[:, _D:] = vq[:, _D:] + contrib


def kernel(inf_query, inf_reference, veh_query, veh_reference, veh_pred_dims,
           veh_scores, veh2inf_rt, W_align, b_align, W_align_pos, b_align_pos,
           W_fusion, b_fusion):
    n_inf = inf_query.shape[0]
    n_veh = veh_query.shape[0]

    inf_pts = _denorm(inf_reference, _INF_PC_RANGE)
    calib = _inv4(veh2inf_rt[0].T)
    homog = jnp.concatenate([inf_pts, jnp.ones_like(inf_pts[:, :1])], axis=-1)
    inf_ptsT = (homog @ calib.T)[:, :3].T
    r9 = calib[:3, :3].reshape(1, 9)
    bp_eff = r9 @ W_align_pos[_D:] + b_align_pos[None]
    bf_eff = r9 @ W_align[_D:] + b_align[None]

    bq = 512
    aligned, fused = pl.pallas_call(
        _align_body,
        grid=(n_inf // bq,),
        in_specs=[
            pl.BlockSpec((bq, 2 * _D), lambda i: (i, 0)),
            pl.BlockSpec((_D, _D), lambda i: (0, 0)),
            pl.BlockSpec((_D, _D), lambda i: (0, 0)),
            pl.BlockSpec((_D, _D), lambda i: (0, 0)),
            pl.BlockSpec((1, _D), lambda i: (0, 0)),
            pl.BlockSpec((1, _D), lambda i: (0, 0)),
            pl.BlockSpec((1, _D), lambda i: (0, 0)),
        ],
        out_specs=[
            pl.BlockSpec((bq, 2 * _D), lambda i: (i, 0)),
            pl.BlockSpec((bq, _D), lambda i: (i, 0)),
        ],
        out_shape=[
            jax.ShapeDtypeStruct((n_inf, 2 * _D), jnp.float32),
            jax.ShapeDtypeStruct((n_inf, _D), jnp.float32),
        ],
    )(inf_query, W_align_pos[:_D], W_align[:_D], W_fusion,
      bp_eff, bf_eff, b_fusion[None])

    bv = 1024
    best_idx, _best_val = pl.pallas_call(
        functools.partial(_match_body, bv=bv, n_inf=n_inf),
        grid=(n_veh // bv,),
        in_specs=[
            pl.BlockSpec((bv, 3), lambda i: (i, 0)),
            pl.BlockSpec((bv, 3), lambda i: (i, 0)),
            pl.BlockSpec((bv, 3), lambda i: (i, 0)),
            pl.BlockSpec((3, n_inf), lambda i: (0, 0)),
        ],
        out_specs=[
            pl.BlockSpec((1, n_inf), lambda i: (0, 0)),
            pl.BlockSpec((1, n_inf), lambda i: (0, 0)),
        ],
        out_shape=[
            jax.ShapeDtypeStruct((1, n_inf), jnp.int32),
            jax.ShapeDtypeStruct((1, n_inf), jnp.float32),
        ],
    )(veh_reference, veh_pred_dims, veh_scores, inf_ptsT)

    bs = 512
    veh_out = pl.pallas_call(
        functools.partial(_scatter_body, bv=bs, n_inf=n_inf),
        grid=(n_veh // bs,),
        in_specs=[
            pl.BlockSpec((bs, 2 * _D), lambda i: (i, 0)),
            pl.BlockSpec((1, n_inf), lambda i: (0, 0)),
            pl.BlockSpec((n_inf, _D), lambda i: (0, 0)),
        ],
        out_specs=pl.BlockSpec((bs, 2 * _D), lambda i: (i, 0)),
        out_shape=jax.ShapeDtypeStruct((n_veh, 2 * _D), jnp.float32),
    )(veh_query, best_idx, fused)

    return veh_out, aligned


# speedup vs baseline: 1.0121x; 1.0104x over previous
import functools

import jax
import jax.numpy as jnp
from jax.experimental import pallas as pl
from jax.experimental.pallas import tpu as pltpu

_D = 256
_PC_RANGE = (-51.2, -51.2, -5.0, 51.2, 51.2, 3.0)
_INF_PC_RANGE = (-76.8, -76.8, -5.0, 76.8, 76.8, 3.0)

_BIG = 1e12
_ACC_THRESH = 1e10


def _denorm(pts, pr):
    x = pts[:, 0:1] * (pr[3] - pr[0]) + pr[0]
    y = pts[:, 1:2] * (pr[4] - pr[1]) + pr[1]
    z = pts[:, 2:3] * (pr[5] - pr[2]) + pr[2]
    return jnp.concatenate([x, y, z], axis=-1)


def _inv4(m):
    rows = []
    idx = (0, 1, 2, 3)
    for r in idx:
        row = []
        for c in idx:
            rs = [i for i in idx if i != r]
            cs = [j for j in idx if j != c]
            det3 = (
                m[rs[0], cs[0]] * (m[rs[1], cs[1]] * m[rs[2], cs[2]]
                                   - m[rs[1], cs[2]] * m[rs[2], cs[1]])
                - m[rs[0], cs[1]] * (m[rs[1], cs[0]] * m[rs[2], cs[2]]
                                     - m[rs[1], cs[2]] * m[rs[2], cs[0]])
                + m[rs[0], cs[2]] * (m[rs[1], cs[0]] * m[rs[2], cs[1]]
                                     - m[rs[1], cs[1]] * m[rs[2], cs[0]])
            )
            row.append(((-1.0) ** (r + c)) * det3)
        rows.append(jnp.stack(row))
    cof = jnp.stack(rows)
    det = jnp.sum(m[0] * cof[0])
    return cof.T / det


def _align_body(q_ref, wp_ref, wf_ref, wfus_ref, bp_ref, bf_ref, bfus_ref,
                aligned_ref, fused_ref):
    q = q_ref[...]
    pos = jnp.dot(q[:, :_D], wp_ref[...],
                  preferred_element_type=jnp.float32) + bp_ref[...]
    feat = jnp.dot(q[:, _D:], wf_ref[...],
                   preferred_element_type=jnp.float32) + bf_ref[...]
    aligned_ref[:, :_D] = pos
    aligned_ref[:, _D:] = feat
    fused_ref[...] = jnp.dot(feat, wfus_ref[...],
                             preferred_element_type=jnp.float32) + bfus_ref[...]


def _match_body(vr_ref, vd_ref, vs_ref, infT_ref, idx_ref, val_ref,
                *, bv, n_inf):
    i = pl.program_id(0)

    @pl.when(i == 0)
    def _():
        val_ref[...] = jnp.full((1, n_inf), _BIG, jnp.float32)
        idx_ref[...] = jnp.full((1, n_inf), -1, jnp.int32)

    infx = infT_ref[0:1, :]
    infy = infT_ref[1:2, :]
    infz = infT_ref[2:3, :]
    sx = _PC_RANGE[3] - _PC_RANGE[0]
    sy = _PC_RANGE[4] - _PC_RANGE[1]
    sz = _PC_RANGE[5] - _PC_RANGE[2]
    vr = vr_ref[...]
    vd = vd_ref[...]
    vs = vs_ref[...]
    ok = jnp.maximum(jnp.maximum(vs[:, 0:1], vs[:, 1:2]), vs[:, 2:3]) >= 0.05
    dx = (vr[:, 0:1] * sx + _PC_RANGE[0]) - infx
    dy = (vr[:, 1:2] * sy + _PC_RANGE[1]) - infy
    dz = (vr[:, 2:3] * sz + _PC_RANGE[2]) - infz
    within = ((jnp.abs(dx) <= jnp.where(ok, vd[:, 0:1], -1.0))
              & (jnp.abs(dy) <= vd[:, 1:2])
              & (jnp.abs(dz) <= vd[:, 2:3]))
    dist2 = dx * dx + dy * dy + dz * dz
    cost = jnp.where(within, dist2, _BIG)
    bmin = jnp.min(cost, axis=0, keepdims=True)
    rows = jax.lax.broadcasted_iota(jnp.int32, (bv, n_inf), 0) + i * bv
    barg = jnp.min(jnp.where(cost == bmin, rows, jnp.int32(2 ** 30)),
                   axis=0, keepdims=True)
    cur = val_ref[...]
    upd = bmin < cur
    val_ref[...] = jnp.where(upd, bmin, cur)
    idx_ref[...] = jnp.where(upd, barg, idx_ref[...])

    @pl.when(i == pl.num_programs(0) - 1)
    def _():
        idx_ref[...] = jnp.where(val_ref[...] < _ACC_THRESH, idx_ref[...], -1)


def _scatter_body(vq_ref, idx_ref, fused_ref, out_ref, *, bv, n_inf):
    i = pl.program_id(0)
    vq = vq_ref[...]
    rows = jax.lax.broadcasted_iota(jnp.int32, (bv, n_inf), 0) + i * bv
    onehot = (rows == idx_ref[...]).astype(jnp.float32)
    contrib = jnp.dot(onehot, fused_ref[...],
                      preferred_element_type=jnp.float32)
    out_ref[:, :_D] = vq[:, :_D]
    out_ref[:, _D:] = vq[:, _D:] + contrib


def kernel(inf_query, inf_reference, veh_query, veh_reference, veh_pred_dims,
           veh_scores, veh2inf_rt, W_align, b_align, W_align_pos, b_align_pos,
           W_fusion, b_fusion):
    n_inf = inf_query.shape[0]
    n_veh = veh_query.shape[0]

    inf_pts = _denorm(inf_reference, _INF_PC_RANGE)
    calib = _inv4(veh2inf_rt[0].T)
    homog = jnp.concatenate([inf_pts, jnp.ones_like(inf_pts[:, :1])], axis=-1)
    inf_ptsT = (homog @ calib.T)[:, :3].T
    r9 = calib[:3, :3].reshape(1, 9)
    bp_eff = r9 @ W_align_pos[_D:] + b_align_pos[None]
    bf_eff = r9 @ W_align[_D:] + b_align[None]

    bq = 512
    aligned, fused = pl.pallas_call(
        _align_body,
        grid=(n_inf // bq,),
        in_specs=[
            pl.BlockSpec((bq, 2 * _D), lambda i: (i, 0)),
            pl.BlockSpec((_D, _D), lambda i: (0, 0)),
            pl.BlockSpec((_D, _D), lambda i: (0, 0)),
            pl.BlockSpec((_D, _D), lambda i: (0, 0)),
            pl.BlockSpec((1, _D), lambda i: (0, 0)),
            pl.BlockSpec((1, _D), lambda i: (0, 0)),
            pl.BlockSpec((1, _D), lambda i: (0, 0)),
        ],
        out_specs=[
            pl.BlockSpec((bq, 2 * _D), lambda i: (i, 0)),
            pl.BlockSpec((bq, _D), lambda i: (i, 0)),
        ],
        out_shape=[
            jax.ShapeDtypeStruct((n_inf, 2 * _D), jnp.float32),
            jax.ShapeDtypeStruct((n_inf, _D), jnp.float32),
        ],
    )(inf_query, W_align_pos[:_D], W_align[:_D], W_fusion,
      bp_eff, bf_eff, b_fusion[None])

    bv = 256
    best_idx, _best_val = pl.pallas_call(
        functools.partial(_match_body, bv=bv, n_inf=n_inf),
        grid=(n_veh // bv,),
        in_specs=[
            pl.BlockSpec((bv, 3), lambda i: (i, 0)),
            pl.BlockSpec((bv, 3), lambda i: (i, 0)),
            pl.BlockSpec((bv, 3), lambda i: (i, 0)),
            pl.BlockSpec((3, n_inf), lambda i: (0, 0)),
        ],
        out_specs=[
            pl.BlockSpec((1, n_inf), lambda i: (0, 0)),
            pl.BlockSpec((1, n_inf), lambda i: (0, 0)),
        ],
        out_shape=[
            jax.ShapeDtypeStruct((1, n_inf), jnp.int32),
            jax.ShapeDtypeStruct((1, n_inf), jnp.float32),
        ],
    )(veh_reference, veh_pred_dims, veh_scores, inf_ptsT)

    bs = 512
    veh_out = pl.pallas_call(
        functools.partial(_scatter_body, bv=bs, n_inf=n_inf),
        grid=(n_veh // bs,),
        in_specs=[
            pl.BlockSpec((bs, 2 * _D), lambda i: (i, 0)),
            pl.BlockSpec((1, n_inf), lambda i: (0, 0)),
            pl.BlockSpec((n_inf, _D), lambda i: (0, 0)),
        ],
        out_specs=pl.BlockSpec((bs, 2 * _D), lambda i: (i, 0)),
        out_shape=jax.ShapeDtypeStruct((n_veh, 2 * _D), jnp.float32),
    )(veh_query, best_idx, fused)

    return veh_out, aligned


# speedup vs baseline: 1.5834x; 1.5645x over previous
import functools

import jax
import jax.numpy as jnp
from jax.experimental import pallas as pl
from jax.experimental.pallas import tpu as pltpu

_D = 256
_PC_RANGE = (-51.2, -51.2, -5.0, 51.2, 51.2, 3.0)
_INF_PC_RANGE = (-76.8, -76.8, -5.0, 76.8, 76.8, 3.0)

_BIG = 1e12
_ACC_THRESH = 1e10


def _denorm(pts, pr):
    x = pts[:, 0:1] * (pr[3] - pr[0]) + pr[0]
    y = pts[:, 1:2] * (pr[4] - pr[1]) + pr[1]
    z = pts[:, 2:3] * (pr[5] - pr[2]) + pr[2]
    return jnp.concatenate([x, y, z], axis=-1)


def _inv4(m):
    rows = []
    idx = (0, 1, 2, 3)
    for r in idx:
        row = []
        for c in idx:
            rs = [i for i in idx if i != r]
            cs = [j for j in idx if j != c]
            det3 = (
                m[rs[0], cs[0]] * (m[rs[1], cs[1]] * m[rs[2], cs[2]]
                                   - m[rs[1], cs[2]] * m[rs[2], cs[1]])
                - m[rs[0], cs[1]] * (m[rs[1], cs[0]] * m[rs[2], cs[2]]
                                     - m[rs[1], cs[2]] * m[rs[2], cs[0]])
                + m[rs[0], cs[2]] * (m[rs[1], cs[0]] * m[rs[2], cs[1]]
                                     - m[rs[1], cs[1]] * m[rs[2], cs[0]])
            )
            row.append(((-1.0) ** (r + c)) * det3)
        rows.append(jnp.stack(row))
    cof = jnp.stack(rows)
    det = jnp.sum(m[0] * cof[0])
    return cof.T / det


def _align_body(q_ref, wp_ref, wf_ref, wfus_ref, bp_ref, bf_ref, bfus_ref,
                aligned_ref, fused_ref):
    q = q_ref[...]
    pos = jnp.dot(q[:, :_D], wp_ref[...],
                  preferred_element_type=jnp.float32) + bp_ref[...]
    feat = jnp.dot(q[:, _D:], wf_ref[...],
                   preferred_element_type=jnp.float32) + bf_ref[...]
    aligned_ref[:, :_D] = pos
    aligned_ref[:, _D:] = feat
    fused_ref[...] = jnp.dot(feat, wfus_ref[...],
                             preferred_element_type=jnp.float32) + bfus_ref[...]


def _match_body(vr_ref, vd_ref, vs_ref, infT_ref, idx_ref, val_ref,
                *, bv, n_inf):
    i = pl.program_id(0)

    @pl.when(i == 0)
    def _():
        val_ref[...] = jnp.full((1, n_inf), _BIG, jnp.float32)
        idx_ref[...] = jnp.full((1, n_inf), -1, jnp.int32)

    infx = infT_ref[0:1, :]
    infy = infT_ref[1:2, :]
    infz = infT_ref[2:3, :]
    sx = _PC_RANGE[3] - _PC_RANGE[0]
    sy = _PC_RANGE[4] - _PC_RANGE[1]
    sz = _PC_RANGE[5] - _PC_RANGE[2]
    vr = vr_ref[...]
    vd = vd_ref[...]
    vs = vs_ref[...]
    ok = jnp.maximum(jnp.maximum(vs[:, 0:1], vs[:, 1:2]), vs[:, 2:3]) >= 0.05
    dx = (vr[:, 0:1] * sx + _PC_RANGE[0]) - infx
    dy = (vr[:, 1:2] * sy + _PC_RANGE[1]) - infy
    dz = (vr[:, 2:3] * sz + _PC_RANGE[2]) - infz
    within = ((jnp.abs(dx) <= jnp.where(ok, vd[:, 0:1], -1.0))
              & (jnp.abs(dy) <= vd[:, 1:2])
              & (jnp.abs(dz) <= vd[:, 2:3]))
    dist2 = dx * dx + dy * dy + dz * dz
    cost = jnp.where(within, dist2, _BIG)
    bmin = jnp.min(cost, axis=0, keepdims=True)
    rows = jax.lax.broadcasted_iota(jnp.int32, (bv, n_inf), 0) + i * bv
    barg = jnp.min(jnp.where(cost == bmin, rows, jnp.int32(2 ** 30)),
                   axis=0, keepdims=True)
    cur = val_ref[...]
    upd = bmin < cur
    val_ref[...] = jnp.where(upd, bmin, cur)
    idx_ref[...] = jnp.where(upd, barg, idx_ref[...])

    @pl.when(i == pl.num_programs(0) - 1)
    def _():
        idx_ref[...] = jnp.where(val_ref[...] < _ACC_THRESH, idx_ref[...], -1)


def _scatter_body(vq_ref, idx_ref, fused_ref, out_ref, *, bv, n_inf):
    i = pl.program_id(0)
    vq = vq_ref[...]
    rows = jax.lax.broadcasted_iota(jnp.int32, (bv, n_inf), 0) + i * bv
    onehot = (rows == idx_ref[...]).astype(jnp.float32)
    contrib = jnp.dot(onehot, fused_ref[...],
                      preferred_element_type=jnp.float32)
    out_ref[:, :_D] = vq[:, :_D]
    out_ref[:, _D:] = vq[:, _D:] + contrib


def kernel(inf_query, inf_reference, veh_query, veh_reference, veh_pred_dims,
           veh_scores, veh2inf_rt, W_align, b_align, W_align_pos, b_align_pos,
           W_fusion, b_fusion):
    n_inf = inf_query.shape[0]
    n_veh = veh_query.shape[0]

    inf_pts = _denorm(inf_reference, _INF_PC_RANGE)
    calib = jnp.linalg.inv(veh2inf_rt[0].T)
    homog = jnp.concatenate([inf_pts, jnp.ones_like(inf_pts[:, :1])], axis=-1)
    inf_ptsT = (homog @ calib.T)[:, :3].T
    r9 = calib[:3, :3].reshape(1, 9)
    bp_eff = r9 @ W_align_pos[_D:] + b_align_pos[None]
    bf_eff = r9 @ W_align[_D:] + b_align[None]

    bq = 512
    aligned, fused = pl.pallas_call(
        _align_body,
        grid=(n_inf // bq,),
        in_specs=[
            pl.BlockSpec((bq, 2 * _D), lambda i: (i, 0)),
            pl.BlockSpec((_D, _D), lambda i: (0, 0)),
            pl.BlockSpec((_D, _D), lambda i: (0, 0)),
            pl.BlockSpec((_D, _D), lambda i: (0, 0)),
            pl.BlockSpec((1, _D), lambda i: (0, 0)),
            pl.BlockSpec((1, _D), lambda i: (0, 0)),
            pl.BlockSpec((1, _D), lambda i: (0, 0)),
        ],
        out_specs=[
            pl.BlockSpec((bq, 2 * _D), lambda i: (i, 0)),
            pl.BlockSpec((bq, _D), lambda i: (i, 0)),
        ],
        out_shape=[
            jax.ShapeDtypeStruct((n_inf, 2 * _D), jnp.float32),
            jax.ShapeDtypeStruct((n_inf, _D), jnp.float32),
        ],
    )(inf_query, W_align_pos[:_D], W_align[:_D], W_fusion,
      bp_eff, bf_eff, b_fusion[None])

    bv = 256
    best_idx, _best_val = pl.pallas_call(
        functools.partial(_match_body, bv=bv, n_inf=n_inf),
        grid=(n_veh // bv,),
        in_specs=[
            pl.BlockSpec((bv, 3), lambda i: (i, 0)),
            pl.BlockSpec((bv, 3), lambda i: (i, 0)),
            pl.BlockSpec((bv, 3), lambda i: (i, 0)),
            pl.BlockSpec((3, n_inf), lambda i: (0, 0)),
        ],
        out_specs=[
            pl.BlockSpec((1, n_inf), lambda i: (0, 0)),
            pl.BlockSpec((1, n_inf), lambda i: (0, 0)),
        ],
        out_shape=[
            jax.ShapeDtypeStruct((1, n_inf), jnp.int32),
            jax.ShapeDtypeStruct((1, n_inf), jnp.float32),
        ],
    )(veh_reference, veh_pred_dims, veh_scores, inf_ptsT)

    bs = 512
    veh_out = pl.pallas_call(
        functools.partial(_scatter_body, bv=bs, n_inf=n_inf),
        grid=(n_veh // bs,),
        in_specs=[
            pl.BlockSpec((bs, 2 * _D), lambda i: (i, 0)),
            pl.BlockSpec((1, n_inf), lambda i: (0, 0)),
            pl.BlockSpec((n_inf, _D), lambda i: (0, 0)),
        ],
        out_specs=pl.BlockSpec((bs, 2 * _D), lambda i: (i, 0)),
        out_shape=jax.ShapeDtypeStruct((n_veh, 2 * _D), jnp.float32),
    )(veh_query, best_idx, fused)

    return veh_out, aligned


# speedup vs baseline: 1.5947x; 1.0072x over previous
import functools

import jax
import jax.numpy as jnp
from jax.experimental import pallas as pl
from jax.experimental.pallas import tpu as pltpu

_D = 256
_PC_RANGE = (-51.2, -51.2, -5.0, 51.2, 51.2, 3.0)
_INF_PC_RANGE = (-76.8, -76.8, -5.0, 76.8, 76.8, 3.0)

_BIG = 1e12
_ACC_THRESH = 1e10


def _denorm(pts, pr):
    x = pts[:, 0:1] * (pr[3] - pr[0]) + pr[0]
    y = pts[:, 1:2] * (pr[4] - pr[1]) + pr[1]
    z = pts[:, 2:3] * (pr[5] - pr[2]) + pr[2]
    return jnp.concatenate([x, y, z], axis=-1)


def _inv4(m):
    rows = []
    idx = (0, 1, 2, 3)
    for r in idx:
        row = []
        for c in idx:
            rs = [i for i in idx if i != r]
            cs = [j for j in idx if j != c]
            det3 = (
                m[rs[0], cs[0]] * (m[rs[1], cs[1]] * m[rs[2], cs[2]]
                                   - m[rs[1], cs[2]] * m[rs[2], cs[1]])
                - m[rs[0], cs[1]] * (m[rs[1], cs[0]] * m[rs[2], cs[2]]
                                     - m[rs[1], cs[2]] * m[rs[2], cs[0]])
                + m[rs[0], cs[2]] * (m[rs[1], cs[0]] * m[rs[2], cs[1]]
                                     - m[rs[1], cs[1]] * m[rs[2], cs[0]])
            )
            row.append(((-1.0) ** (r + c)) * det3)
        rows.append(jnp.stack(row))
    cof = jnp.stack(rows)
    det = jnp.sum(m[0] * cof[0])
    return cof.T / det


def _align_body(q_ref, wp_ref, wf_ref, wfus_ref, bp_ref, bf_ref, bfus_ref,
                aligned_ref, fused_ref):
    q = q_ref[...]
    pos = jnp.dot(q[:, :_D], wp_ref[...],
                  preferred_element_type=jnp.float32) + bp_ref[...]
    feat = jnp.dot(q[:, _D:], wf_ref[...],
                   preferred_element_type=jnp.float32) + bf_ref[...]
    aligned_ref[:, :_D] = pos
    aligned_ref[:, _D:] = feat
    fused_ref[...] = jnp.dot(feat, wfus_ref[...],
                             preferred_element_type=jnp.float32) + bfus_ref[...]


def _match_body(vr_ref, vd_ref, vs_ref, infT_ref, idx_ref, val_ref,
                *, bv, n_inf):
    i = pl.program_id(0)

    @pl.when(i == 0)
    def _():
        val_ref[...] = jnp.full((1, n_inf), _BIG, jnp.float32)
        idx_ref[...] = jnp.full((1, n_inf), -1, jnp.int32)

    infx = infT_ref[0:1, :]
    infy = infT_ref[1:2, :]
    infz = infT_ref[2:3, :]
    sx = _PC_RANGE[3] - _PC_RANGE[0]
    sy = _PC_RANGE[4] - _PC_RANGE[1]
    sz = _PC_RANGE[5] - _PC_RANGE[2]
    vr = vr_ref[...]
    vd = vd_ref[...]
    vs = vs_ref[...]
    ok = jnp.maximum(jnp.maximum(vs[:, 0:1], vs[:, 1:2]), vs[:, 2:3]) >= 0.05
    dx = (vr[:, 0:1] * sx + _PC_RANGE[0]) - infx
    dy = (vr[:, 1:2] * sy + _PC_RANGE[1]) - infy
    dz = (vr[:, 2:3] * sz + _PC_RANGE[2]) - infz
    within = ((jnp.abs(dx) <= jnp.where(ok, vd[:, 0:1], -1.0))
              & (jnp.abs(dy) <= vd[:, 1:2])
              & (jnp.abs(dz) <= vd[:, 2:3]))
    dist2 = dx * dx + dy * dy + dz * dz
    cost = jnp.where(within, dist2, _BIG)
    bmin = jnp.min(cost, axis=0, keepdims=True)
    rows = jax.lax.broadcasted_iota(jnp.int32, (bv, n_inf), 0) + i * bv
    barg = jnp.min(jnp.where(cost == bmin, rows, jnp.int32(2 ** 30)),
                   axis=0, keepdims=True)
    cur = val_ref[...]
    upd = bmin < cur
    val_ref[...] = jnp.where(upd, bmin, cur)
    idx_ref[...] = jnp.where(upd, barg, idx_ref[...])

    @pl.when(i == pl.num_programs(0) - 1)
    def _():
        idx_ref[...] = jnp.where(val_ref[...] < _ACC_THRESH, idx_ref[...], -1)


def _scatter_body(vq_ref, idx_ref, fused_ref, out_ref, *, bv, n_inf):
    i = pl.program_id(0)
    vq = vq_ref[...]
    rows = jax.lax.broadcasted_iota(jnp.int32, (bv, n_inf), 0) + i * bv
    onehot = (rows == idx_ref[...]).astype(jnp.float32)
    contrib = jnp.dot(onehot, fused_ref[...],
                      preferred_element_type=jnp.float32)
    out_ref[:, :_D] = vq[:, :_D]
    out_ref[:, _D:] = vq[:, _D:] + contrib


def kernel(inf_query, inf_reference, veh_query, veh_reference, veh_pred_dims,
           veh_scores, veh2inf_rt, W_align, b_align, W_align_pos, b_align_pos,
           W_fusion, b_fusion):
    n_inf = inf_query.shape[0]
    n_veh = veh_query.shape[0]

    inf_pts = _denorm(inf_reference, _INF_PC_RANGE)
    calib = jnp.linalg.inv(veh2inf_rt[0].T)
    homog = jnp.concatenate([inf_pts, jnp.ones_like(inf_pts[:, :1])], axis=-1)
    inf_ptsT = (homog @ calib.T)[:, :3].T
    r9 = calib[:3, :3].reshape(1, 9)
    bp_eff = r9 @ W_align_pos[_D:] + b_align_pos[None]
    bf_eff = r9 @ W_align[_D:] + b_align[None]

    bq = 512
    aligned, fused = pl.pallas_call(
        _align_body,
        grid=(n_inf // bq,),
        in_specs=[
            pl.BlockSpec((bq, 2 * _D), lambda i: (i, 0)),
            pl.BlockSpec((_D, _D), lambda i: (0, 0)),
            pl.BlockSpec((_D, _D), lambda i: (0, 0)),
            pl.BlockSpec((_D, _D), lambda i: (0, 0)),
            pl.BlockSpec((1, _D), lambda i: (0, 0)),
            pl.BlockSpec((1, _D), lambda i: (0, 0)),
            pl.BlockSpec((1, _D), lambda i: (0, 0)),
        ],
        out_specs=[
            pl.BlockSpec((bq, 2 * _D), lambda i: (i, 0)),
            pl.BlockSpec((bq, _D), lambda i: (i, 0)),
        ],
        out_shape=[
            jax.ShapeDtypeStruct((n_inf, 2 * _D), jnp.float32),
            jax.ShapeDtypeStruct((n_inf, _D), jnp.float32),
        ],
    )(inf_query, W_align_pos[:_D], W_align[:_D], W_fusion,
      bp_eff, bf_eff, b_fusion[None])

    bv = 512
    best_idx, _best_val = pl.pallas_call(
        functools.partial(_match_body, bv=bv, n_inf=n_inf),
        grid=(n_veh // bv,),
        in_specs=[
            pl.BlockSpec((bv, 3), lambda i: (i, 0)),
            pl.BlockSpec((bv, 3), lambda i: (i, 0)),
            pl.BlockSpec((bv, 3), lambda i: (i, 0)),
            pl.BlockSpec((3, n_inf), lambda i: (0, 0)),
        ],
        out_specs=[
            pl.BlockSpec((1, n_inf), lambda i: (0, 0)),
            pl.BlockSpec((1, n_inf), lambda i: (0, 0)),
        ],
        out_shape=[
            jax.ShapeDtypeStruct((1, n_inf), jnp.int32),
            jax.ShapeDtypeStruct((1, n_inf), jnp.float32),
        ],
    )(veh_reference, veh_pred_dims, veh_scores, inf_ptsT)

    bs = 512
    veh_out = pl.pallas_call(
        functools.partial(_scatter_body, bv=bs, n_inf=n_inf),
        grid=(n_veh // bs,),
        in_specs=[
            pl.BlockSpec((bs, 2 * _D), lambda i: (i, 0)),
            pl.BlockSpec((1, n_inf), lambda i: (0, 0)),
            pl.BlockSpec((n_inf, _D), lambda i: (0, 0)),
        ],
        out_specs=pl.BlockSpec((bs, 2 * _D), lambda i: (i, 0)),
        out_shape=jax.ShapeDtypeStruct((n_veh, 2 * _D), jnp.float32),
    )(veh_query, best_idx, fused)

    return veh_out, aligned
